# Initial kernel scaffold; baseline (speedup 1.0000x reference)
import jax
import jax.numpy as jnp
from jax.experimental import pallas as pl


def kernel(x, table, sum_over_quantizers):
    raise NotImplementedError("write your pallas kernel here")


# speedup vs baseline: 1.0208x; 1.0208x over previous
import functools

import jax
import jax.numpy as jnp
from jax import lax
from jax.experimental import pallas as pl
from jax.experimental.pallas import tpu as pltpu
from jax.experimental.pallas import tpu_sc as plsc

LANES = 16
IDX_PER_DMA = 128


@functools.lru_cache(maxsize=None)
def _build(n_tok: int, q: int, f: int):
    info = plsc.get_sparse_core_info()
    nc, ns = info.num_cores, info.num_subcores
    nw = nc * ns

    assert f % LANES == 0
    assert n_tok % nw == 0
    tok_w = n_tok // nw
    t_chunk = 64
    while tok_w % t_chunk:
        t_chunk //= 2
    n_chunks = tok_w // t_chunk
    rows_chunk = t_chunk * q
    assert rows_chunk % IDX_PER_DMA == 0
    n_sub = rows_chunk // IDX_PER_DMA
    f_v = f // LANES

    mesh = plsc.VectorSubcoreMesh(core_axis_name="c", subcore_axis_name="s")

    @functools.partial(
        pl.kernel,
        out_type=jax.ShapeDtypeStruct((n_tok, f), jnp.float32),
        mesh=mesh,
        compiler_params=pltpu.CompilerParams(use_tc_tiling_on_sc=False),
        scratch_types=[
            pltpu.VMEM((rows_chunk,), jnp.int32),
            pltpu.VMEM((rows_chunk, f), jnp.float32),
            pltpu.VMEM((t_chunk, f), jnp.float32),
            pltpu.VMEM((LANES,), jnp.int32),
            pltpu.SemaphoreType.DMA,
        ],
    )
    def emb(xf_hbm, table_hbm, off_hbm, out_hbm, idx_v, rows_v, out_v, off_v, gsem):
        wid = lax.axis_index("s") * nc + lax.axis_index("c")
        base_t = wid * tok_w

        pltpu.sync_copy(off_hbm, off_v)
        offv = off_v[...]

        def chunk_body(g, carry):
            t0 = base_t + g * t_chunk
            pltpu.sync_copy(xf_hbm.at[pl.ds(t0 * q, rows_chunk)], idx_v)

            def off_body(j, c):
                s = pl.multiple_of(j * LANES, LANES)
                idx_v[pl.ds(s, LANES)] = idx_v[pl.ds(s, LANES)] + offv
                return c

            lax.fori_loop(0, rows_chunk // LANES, off_body, 0)

            handles = []
            for j in range(n_sub):
                h = pltpu.async_copy(
                    table_hbm.at[idx_v.at[pl.ds(j * IDX_PER_DMA, IDX_PER_DMA)]],
                    rows_v.at[pl.ds(j * IDX_PER_DMA, IDX_PER_DMA)],
                    gsem,
                )
                handles.append(h)
            for h in handles:
                h.wait()

            def tok_body(t, c):
                rbase = t * q
                for cc in range(f_v):
                    sl = pl.ds(cc * LANES, LANES)
                    acc = rows_v[rbase, sl]
                    for qq in range(1, q):
                        acc = acc + rows_v[rbase + qq, sl]
                    out_v[t, sl] = acc
                return c

            lax.fori_loop(0, t_chunk, tok_body, 0)

            pltpu.sync_copy(out_v, out_hbm.at[pl.ds(t0, t_chunk)])
            return carry

        lax.fori_loop(0, n_chunks, chunk_body, 0)

    return emb


def kernel(x, table, sum_over_quantizers):
    b, s, q = x.shape
    v, f = table.shape
    seg = v // q
    n_tok = b * s

    flag = jnp.asarray(sum_over_quantizers).astype(jnp.int32)
    off16 = (jnp.arange(LANES, dtype=jnp.int32) % q) * jnp.int32(seg) * flag

    xf = x.reshape(n_tok * q)
    out = _build(n_tok, q, f)(xf, table, off16)
    return out.reshape(b, s, f)


# speedup vs baseline: 1.1973x; 1.1729x over previous
import functools

import jax
import jax.numpy as jnp
from jax import lax
from jax.experimental import pallas as pl
from jax.experimental.pallas import tpu as pltpu
from jax.experimental.pallas import tpu_sc as plsc

LANES = 16
IDX_PER_DMA = 128
NBUF = 2


@functools.lru_cache(maxsize=None)
def _build(n_tok: int, q: int, f: int):
    info = plsc.get_sparse_core_info()
    nc, ns = info.num_cores, info.num_subcores
    nw = nc * ns

    assert f % LANES == 0
    assert n_tok % nw == 0
    tok_w = n_tok // nw
    t_chunk = 64
    while tok_w % (t_chunk * NBUF):
        t_chunk //= 2
    n_chunks = tok_w // t_chunk
    rows_chunk = t_chunk * q
    assert rows_chunk % IDX_PER_DMA == 0
    n_sub = rows_chunk // IDX_PER_DMA
    f_v = f // LANES

    mesh = plsc.VectorSubcoreMesh(core_axis_name="c", subcore_axis_name="s")

    @functools.partial(
        pl.kernel,
        out_type=jax.ShapeDtypeStruct((n_tok, f), jnp.float32),
        mesh=mesh,
        compiler_params=pltpu.CompilerParams(use_tc_tiling_on_sc=False),
        scratch_types=[
            pltpu.VMEM((NBUF, rows_chunk), jnp.int32),
            pltpu.VMEM((NBUF, rows_chunk, f), jnp.float32),
            pltpu.VMEM((NBUF, t_chunk, f), jnp.float32),
            pltpu.VMEM((LANES,), jnp.int32),
            pltpu.SemaphoreType.DMA,
            pltpu.SemaphoreType.DMA,
        ],
    )
    def emb(xf_hbm, table_hbm, off_hbm, out_hbm, idx_v, rows_v, out_v, off_v,
            gsem0, gsem1):
        gsems = (gsem0, gsem1)
        wid = lax.axis_index("s") * nc + lax.axis_index("c")
        base_t = wid * tok_w

        pltpu.sync_copy(off_hbm, off_v)
        offv = off_v[...]

        def gather_descs(b):
            return [
                pltpu.make_async_copy(
                    table_hbm.at[idx_v.at[b, pl.ds(j * IDX_PER_DMA, IDX_PER_DMA)]],
                    rows_v.at[b, pl.ds(j * IDX_PER_DMA, IDX_PER_DMA)],
                    gsems[b],
                )
                for j in range(n_sub)
            ]

        def stage(b, g):
            t0 = base_t + g * t_chunk
            pltpu.sync_copy(xf_hbm.at[pl.ds(t0 * q, rows_chunk)], idx_v.at[b])

            def off_body(j, c):
                s = pl.multiple_of(j * LANES, LANES)
                idx_v[b, pl.ds(s, LANES)] = idx_v[b, pl.ds(s, LANES)] + offv
                return c

            lax.fori_loop(0, rows_chunk // LANES, off_body, 0)
            for h in gather_descs(b):
                h.start()

        def compute(b, g):
            for h in gather_descs(b):
                h.wait()

            def tok_body(t, c):
                rbase = t * q
                for cc in range(f_v):
                    sl = pl.ds(cc * LANES, LANES)
                    acc = rows_v[b, rbase, sl]
                    for qq in range(1, q):
                        acc = acc + rows_v[b, rbase + qq, sl]
                    out_v[b, t, sl] = acc
                return c

            lax.fori_loop(0, t_chunk, tok_body, 0)
            pltpu.sync_copy(out_v.at[b], out_hbm.at[pl.ds(base_t + g * t_chunk, t_chunk)])

        for b in range(NBUF):
            stage(b, b)

        def loop_body(i, c):
            gg = i * NBUF
            for b in range(NBUF):
                g = gg + b
                compute(b, g)

                @pl.when(g + NBUF < n_chunks)
                def _():
                    stage(b, g + NBUF)

            return c

        lax.fori_loop(0, n_chunks // NBUF, loop_body, 0)

    return emb


def kernel(x, table, sum_over_quantizers):
    b, s, q = x.shape
    v, f = table.shape
    seg = v // q
    n_tok = b * s

    flag = jnp.asarray(sum_over_quantizers).astype(jnp.int32)
    off16 = (jnp.arange(LANES, dtype=jnp.int32) % q) * jnp.int32(seg) * flag

    xf = x.reshape(n_tok * q)
    out = _build(n_tok, q, f)(xf, table, off16)
    return out.reshape(b, s, f)


# speedup vs baseline: 1.4633x; 1.2221x over previous
import functools

import jax
import jax.numpy as jnp
from jax import lax
from jax.experimental import pallas as pl
from jax.experimental.pallas import tpu as pltpu
from jax.experimental.pallas import tpu_sc as plsc

LANES = 16
IDX_PER_DMA = 128
NBUF = 2


@functools.lru_cache(maxsize=None)
def _build(n_tok: int, q: int, f: int):
    info = plsc.get_sparse_core_info()
    nc, ns = info.num_cores, info.num_subcores
    nw = nc * ns

    assert f % LANES == 0
    assert n_tok % nw == 0
    tok_w = n_tok // nw
    t_chunk = 64
    while tok_w % (t_chunk * NBUF):
        t_chunk //= 2
    n_chunks = tok_w // t_chunk
    rows_chunk = t_chunk * q
    assert rows_chunk % IDX_PER_DMA == 0
    n_sub = rows_chunk // IDX_PER_DMA
    f_v = f // LANES

    mesh = plsc.VectorSubcoreMesh(core_axis_name="c", subcore_axis_name="s")

    @functools.partial(
        pl.kernel,
        out_type=jax.ShapeDtypeStruct((n_tok, f), jnp.float32),
        mesh=mesh,
        compiler_params=pltpu.CompilerParams(use_tc_tiling_on_sc=False),
        scratch_types=[
            pltpu.VMEM((NBUF, rows_chunk), jnp.int32),
            pltpu.VMEM((NBUF, rows_chunk, f), jnp.float32),
            pltpu.VMEM((NBUF, t_chunk, f), jnp.float32),
            pltpu.VMEM((LANES,), jnp.int32),
            pltpu.SemaphoreType.DMA,
            pltpu.SemaphoreType.DMA,
        ],
    )
    def emb(xf_hbm, table_hbm, off_hbm, out_hbm, idx_v, rows_v, out_v, off_v,
            gsem0, gsem1):
        gsems = (gsem0, gsem1)
        wid = lax.axis_index("s") * nc + lax.axis_index("c")
        base_t = wid * tok_w

        pltpu.sync_copy(off_hbm, off_v)
        offv = off_v[...]

        def gather_descs(b):
            return [
                pltpu.make_async_copy(
                    table_hbm.at[idx_v.at[b, pl.ds(j * IDX_PER_DMA, IDX_PER_DMA)]],
                    rows_v.at[b, pl.ds(j * IDX_PER_DMA, IDX_PER_DMA)],
                    gsems[b],
                )
                for j in range(n_sub)
            ]

        def stage(b, g):
            t0 = base_t + g * t_chunk
            pltpu.sync_copy(xf_hbm.at[pl.ds(t0 * q, rows_chunk)], idx_v.at[b])

            @plsc.parallel_loop(0, rows_chunk, LANES, unroll=2)
            def off_body(s):
                idx_v[b, pl.ds(s, LANES)] = idx_v[b, pl.ds(s, LANES)] + offv
            for h in gather_descs(b):
                h.start()

        def compute(b, g):
            for h in gather_descs(b):
                h.wait()

            @plsc.parallel_loop(0, t_chunk, 1, unroll=2)
            def tok_body(t):
                rbase = t * q
                for cc in range(f_v):
                    sl = pl.ds(cc * LANES, LANES)
                    vals = [rows_v[b, rbase + qq, sl] for qq in range(q)]
                    while len(vals) > 1:
                        vals = [vals[i] + vals[i + 1] for i in range(0, len(vals) - 1, 2)] + (
                            [vals[-1]] if len(vals) % 2 else [])
                    out_v[b, t, sl] = vals[0]
            pltpu.sync_copy(out_v.at[b], out_hbm.at[pl.ds(base_t + g * t_chunk, t_chunk)])

        for b in range(NBUF):
            stage(b, b)

        def loop_body(i, c):
            gg = i * NBUF
            for b in range(NBUF):
                g = gg + b
                compute(b, g)

                @pl.when(g + NBUF < n_chunks)
                def _():
                    stage(b, g + NBUF)

            return c

        lax.fori_loop(0, n_chunks // NBUF, loop_body, 0)

    return emb


def kernel(x, table, sum_over_quantizers):
    b, s, q = x.shape
    v, f = table.shape
    seg = v // q
    n_tok = b * s

    flag = jnp.asarray(sum_over_quantizers).astype(jnp.int32)
    off16 = (jnp.arange(LANES, dtype=jnp.int32) % q) * jnp.int32(seg) * flag

    xf = x.reshape(n_tok * q)
    out = _build(n_tok, q, f)(xf, table, off16)
    return out.reshape(b, s, f)
